# baseline (speedup 1.0000x reference)
import functools

import jax
import jax.numpy as jnp
from jax.experimental import pallas as pl
from jax.experimental.pallas import tpu as pltpu


def _gnn_cell_kernel(a_ref, h_ref, u_ref, bias_ref, g0_ref, out_ref, *, bb, n, h):
    f = 3 * h
    h3 = h_ref[...]
    h2 = h3.reshape(bb * n, h)
    p2 = jnp.dot(h2, u_ref[...], preferred_element_type=jnp.float32)
    p2 = p2 + bias_ref[...]
    p3 = p2.reshape(bb, n, 3 * f)
    a = a_ref[...].reshape(bb, n, 2 * n)

    dn = (((2,), (1,)), ((0,), (0,)))
    p_stack = jnp.concatenate([p3[:, :, :f], p3[:, :, f:2 * f]], axis=1)
    gi = jax.lax.dot_general(a, p_stack, dn,
                             preferred_element_type=jnp.float32)
    gi = gi + g0_ref[...].reshape(1, 1, f)

    gh = p3[:, :, 2 * f:]
    resetgate = jax.nn.sigmoid(gi[:, :, :h] + gh[:, :, :h])
    inputgate = jax.nn.sigmoid(gi[:, :, h:2 * h] + gh[:, :, h:2 * h])
    newgate = jnp.tanh(gi[:, :, 2 * h:] + resetgate * gh[:, :, 2 * h:])
    out_ref[...] = h3 + inputgate * (newgate - h3)


def kernel(A, hidden, mask, W_ein, b_ein, W_eout, b_eout, b_iah, b_oah, w_ih, w_hh, b_ih, b_hh):
    b, n, h = hidden.shape
    f = 3 * h
    bb = 64
    assert b % bb == 0

    wt_in = w_ih[:, :h].T
    wt_out = w_ih[:, h:].T
    u_cat = jnp.concatenate([W_ein.T @ wt_in, W_eout.T @ wt_out, w_hh.T], axis=1)
    bias_cat = jnp.concatenate([b_ein @ wt_in, b_eout @ wt_out, b_hh])[None, :]
    gi_const = (b_iah @ wt_in + b_oah @ wt_out + b_ih)[None, :]
    a2 = A.reshape(b, 2 * n * n)

    grid = (b // bb,)
    return pl.pallas_call(
        functools.partial(_gnn_cell_kernel, bb=bb, n=n, h=h),
        grid=grid,
        in_specs=[
            pl.BlockSpec((bb, 2 * n * n), lambda i: (i, 0)),
            pl.BlockSpec((bb, n, h), lambda i: (i, 0, 0)),
            pl.BlockSpec((h, 3 * f), lambda i: (0, 0)),
            pl.BlockSpec((1, 3 * f), lambda i: (0, 0)),
            pl.BlockSpec((1, f), lambda i: (0, 0)),
        ],
        out_specs=pl.BlockSpec((bb, n, h), lambda i: (i, 0, 0)),
        out_shape=jax.ShapeDtypeStruct((b, n, h), jnp.float32),
        compiler_params=pltpu.CompilerParams(
            dimension_semantics=("parallel",)),
    )(a2, hidden, u_cat, bias_cat, gi_const)


# speedup vs baseline: 1.0260x; 1.0260x over previous
import functools

import jax
import jax.numpy as jnp
from jax.experimental import pallas as pl
from jax.experimental.pallas import tpu as pltpu


def _gnn_cell_kernel(a_ref, h_ref, u_ref, bias_ref, g0_ref, out_ref, *, bb, n, h):
    f = 3 * h
    h3 = h_ref[...]
    h2 = h3.reshape(bb * n, h)
    p2 = jnp.dot(h2, u_ref[...], preferred_element_type=jnp.float32)
    p2 = p2 + bias_ref[...]
    p3 = p2.reshape(bb, n, 3 * f)
    a = a_ref[...].reshape(bb, n, 2 * n)

    dn = (((2,), (1,)), ((0,), (0,)))
    p_stack = jnp.concatenate([p3[:, :, :f], p3[:, :, f:2 * f]], axis=1)
    gi = jax.lax.dot_general(a, p_stack, dn,
                             preferred_element_type=jnp.float32)
    gi = gi + g0_ref[...].reshape(1, 1, f)

    gh = p3[:, :, 2 * f:]
    resetgate = jax.nn.sigmoid(gi[:, :, :h] + gh[:, :, :h])
    inputgate = jax.nn.sigmoid(gi[:, :, h:2 * h] + gh[:, :, h:2 * h])
    newgate = jnp.tanh(gi[:, :, 2 * h:] + resetgate * gh[:, :, 2 * h:])
    out_ref[...] = h3 + inputgate * (newgate - h3)


def kernel(A, hidden, mask, W_ein, b_ein, W_eout, b_eout, b_iah, b_oah, w_ih, w_hh, b_ih, b_hh):
    b, n, h = hidden.shape
    f = 3 * h
    bb = 256
    assert b % bb == 0

    wt_in = w_ih[:, :h].T
    wt_out = w_ih[:, h:].T
    u_cat = jnp.concatenate([W_ein.T @ wt_in, W_eout.T @ wt_out, w_hh.T], axis=1)
    bias_cat = jnp.concatenate([b_ein @ wt_in, b_eout @ wt_out, b_hh])[None, :]
    gi_const = (b_iah @ wt_in + b_oah @ wt_out + b_ih)[None, :]
    a2 = A.reshape(b, 2 * n * n)

    grid = (b // bb,)
    return pl.pallas_call(
        functools.partial(_gnn_cell_kernel, bb=bb, n=n, h=h),
        grid=grid,
        in_specs=[
            pl.BlockSpec((bb, 2 * n * n), lambda i: (i, 0)),
            pl.BlockSpec((bb, n, h), lambda i: (i, 0, 0)),
            pl.BlockSpec((h, 3 * f), lambda i: (0, 0)),
            pl.BlockSpec((1, 3 * f), lambda i: (0, 0)),
            pl.BlockSpec((1, f), lambda i: (0, 0)),
        ],
        out_specs=pl.BlockSpec((bb, n, h), lambda i: (i, 0, 0)),
        out_shape=jax.ShapeDtypeStruct((b, n, h), jnp.float32),
        compiler_params=pltpu.CompilerParams(
            dimension_semantics=("parallel",)),
    )(a2, hidden, u_cat, bias_cat, gi_const)
